# Initial kernel scaffold; baseline (speedup 1.0000x reference)
import jax
import jax.numpy as jnp
from jax.experimental import pallas as pl


def kernel(value, query, edge_index, edge_weight, W, b, ln_gamma, ln_beta):
    raise NotImplementedError("write your pallas kernel here")


# speedup vs baseline: 2.9889x; 2.9889x over previous
import functools

import jax
import jax.numpy as jnp
from jax import lax
from jax.experimental import pallas as pl
from jax.experimental.pallas import tpu as pltpu
from jax.experimental.pallas import tpu_sc as plsc

N = 10000
E = 160000
D = 256
DH = D // 2
NS = 16
L = 16
EPT = E // NS
K = 80
NCHUNK = EPT // K
RPT = N // NS


def _sc_spmm_body(q2, src, dst, ew, out, idxv, dstv, wv, rows, sem):
    c = lax.axis_index("c")
    s = lax.axis_index("s")

    zeros16 = jnp.zeros((L,), jnp.float32)

    @pl.loop(0, K)
    def _zero_rows(r):
        for j in range(DH // L):
            rows[r, pl.ds(j * L, L)] = zeros16

    row0 = (s * RPT) // 8 * 8
    for i in range(7):
        pltpu.sync_copy(rows, out.at[pl.ds(row0 + i * K, K)])
    pltpu.sync_copy(rows.at[pl.ds(0, 72)], out.at[pl.ds(row0 + 560, 72)])
    plsc.subcore_barrier()

    ebase = s * EPT

    @pl.loop(0, NCHUNK)
    def _chunk(t):
        base = ebase + t * K
        pltpu.sync_copy(src.at[pl.ds(base, K)], idxv)
        pltpu.sync_copy(dst.at[pl.ds(base, K)], dstv)
        pltpu.sync_copy(ew.at[pl.ds(base, K)], wv)
        for j in range(K // L):
            sl = pl.ds(j * L, L)
            idxv[sl] = idxv[sl] * 2 + c
        pltpu.async_copy(q2.at[idxv], rows, sem).wait()

        @pl.loop(0, K // L)
        def _scale(g):
            w16 = wv[pl.ds(g * L, L)]
            for k16 in range(L):
                wk = w16[k16]
                r = g * L + k16
                for j in range(DH // L):
                    sl = pl.ds(j * L, L)
                    rows[r, sl] = rows[r, sl] * wk

        pltpu.sync_copy(rows, out.at[dstv], add=True)

    plsc.subcore_barrier()


def _sc_copyout_body(acc, out, rows, c, s):
    row0 = (s * RPT) // 8 * 8
    for i in range(7):
        pltpu.sync_copy(acc.at[pl.ds(row0 + i * K, K)], rows)
        pltpu.sync_copy(rows, out.at[c, pl.ds(row0 + i * K, K)])
    pltpu.sync_copy(acc.at[pl.ds(row0 + 560, 72)], rows.at[pl.ds(0, 72)])
    pltpu.sync_copy(rows.at[pl.ds(0, 72)], out.at[c, pl.ds(row0 + 560, 72)])


def _make_sc_spmm():
    mesh = plsc.VectorSubcoreMesh(core_axis_name="c", subcore_axis_name="s")

    @functools.partial(
        pl.kernel,
        out_type=jax.ShapeDtypeStruct((2, N, DH), jnp.float32),
        mesh=mesh,
        scratch_types=[
            pltpu.VMEM_SHARED((N, DH), jnp.float32),
            pltpu.VMEM((K,), jnp.int32),
            pltpu.VMEM((K,), jnp.int32),
            pltpu.VMEM((K,), jnp.float32),
            pltpu.VMEM((K, DH), jnp.float32),
            pltpu.SemaphoreType.DMA,
        ],
    )
    def spmm(q2, src, dst, ew, out, acc, idxv, dstv, wv, rows, sem):
        c = lax.axis_index("c")
        s = lax.axis_index("s")
        _sc_spmm_body(q2, src, dst, ew, acc, idxv, dstv, wv, rows, sem)
        _sc_copyout_body(acc, out, rows, c, s)

    return spmm


_sc_spmm = _make_sc_spmm()


def _tc_mlp_body(x_ref, w_ref, b_ref, g_ref, bt_ref, o_ref):
    xa = x_ref[0]
    xb = x_ref[1]
    y = (jnp.dot(xa, w_ref[0:DH, :], preferred_element_type=jnp.float32)
         + jnp.dot(xb, w_ref[DH:D, :], preferred_element_type=jnp.float32)
         + b_ref[...])
    mean = jnp.mean(y, axis=1, keepdims=True)
    yc = y - mean
    var = jnp.mean(yc * yc, axis=1, keepdims=True)
    yn = yc * lax.rsqrt(var + 1e-5) * g_ref[...] + bt_ref[...]
    o_ref[...] = 0.5 * yn * (1.0 + lax.erf(yn * 0.7071067811865475))


def _tc_mlp(xh, W, b, gamma, beta):
    BN = 1000
    grid = (N // BN,)
    return pl.pallas_call(
        _tc_mlp_body,
        grid=grid,
        in_specs=[
            pl.BlockSpec((2, BN, DH), lambda i: (0, i, 0)),
            pl.BlockSpec((D, D), lambda i: (0, 0)),
            pl.BlockSpec((1, D), lambda i: (0, 0)),
            pl.BlockSpec((1, D), lambda i: (0, 0)),
            pl.BlockSpec((1, D), lambda i: (0, 0)),
        ],
        out_specs=pl.BlockSpec((BN, D), lambda i: (i, 0)),
        out_shape=jax.ShapeDtypeStruct((N, D), jnp.float32),
    )(xh, W, b.reshape(1, D), gamma.reshape(1, D), beta.reshape(1, D))


def kernel(value, query, edge_index, edge_weight, W, b, ln_gamma, ln_beta):
    del value
    src = edge_index[1].astype(jnp.int32)
    dst = edge_index[0].astype(jnp.int32)
    q2 = query.reshape(2 * N, DH)
    xh = _sc_spmm(q2, src, dst, edge_weight.astype(jnp.float32))
    return _tc_mlp(xh, W, b, ln_gamma, ln_beta)
